# baseline (speedup 1.0000x reference)
import functools

import jax
import jax.numpy as jnp
from jax import lax
from jax.experimental import pallas as pl
from jax.experimental.pallas import tpu as pltpu
from jax.experimental.pallas import tpu_sc as plsc

N_FACTORS = 32
BATCH = 16384
NC = 2
NS = 16
NW = NC * NS
B_PER_W = BATCH // NW
CHUNK = 128
N_CHUNKS = B_PER_W // CHUNK
LANES = 16
GROUPS_PER_CHUNK = CHUNK // LANES


def _gmf_body(user_hbm, item_hbm, uemb_hbm, iemb_hbm, hw_hbm, hb_hbm,
              out_hbm, idx_u, idx_i, rows_u, rows_i, w_v, b_v, out_v, sem):
    wid = lax.axis_index("s") * NC + lax.axis_index("c")
    base = wid * B_PER_W

    pltpu.sync_copy(hw_hbm.at[0], w_v)
    pltpu.sync_copy(hb_hbm, b_v.at[pl.ds(0, 1)])

    for c in range(N_CHUNKS):
        pltpu.sync_copy(user_hbm.at[pl.ds(base + c * CHUNK, CHUNK)], idx_u.at[c])
        pltpu.sync_copy(item_hbm.at[pl.ds(base + c * CHUNK, CHUNK)], idx_i.at[c])

    copies = []
    for c in range(N_CHUNKS):
        copies.append(pltpu.async_copy(uemb_hbm.at[idx_u.at[c]], rows_u.at[c], sem))
        copies.append(pltpu.async_copy(iemb_hbm.at[idx_i.at[c]], rows_i.at[c], sem))
    for cp in copies:
        cp.wait()

    w0 = w_v[pl.ds(0, LANES)]
    w1 = w_v[pl.ds(LANES, LANES)]
    b = b_v[pl.ds(0, LANES)][0]
    lane = jnp.arange(LANES, dtype=jnp.int32)

    for c in range(N_CHUNKS):
        def group(g, _, c=c):
            acc = jnp.zeros((LANES,), jnp.float32)
            for j in range(LANES):
                r = g * LANES + j
                p0 = rows_u[c, r, pl.ds(0, LANES)]
                p1 = rows_u[c, r, pl.ds(LANES, LANES)]
                q0 = rows_i[c, r, pl.ds(0, LANES)]
                q1 = rows_i[c, r, pl.ds(LANES, LANES)]
                s = p0 * q0 * w0 + p1 * q1 * w1
                tot = jnp.sum(s)
                acc = jnp.where(lane == j, tot, acc)
            out_v[pl.ds(c * CHUNK + g * LANES, LANES)] = acc + b
            return 0
        lax.fori_loop(0, GROUPS_PER_CHUNK, group, 0)

    pltpu.sync_copy(out_v, out_hbm.at[pl.ds(base, B_PER_W)])


@jax.jit
def _gmf(user, item, user_emb, item_emb, h_w, h_b):
    mesh = plsc.VectorSubcoreMesh(core_axis_name="c", subcore_axis_name="s")
    call = functools.partial(
        pl.kernel,
        mesh=mesh,
        out_type=jax.ShapeDtypeStruct((BATCH,), jnp.float32),
        scratch_types=[
            pltpu.VMEM((N_CHUNKS, CHUNK), jnp.int32),
            pltpu.VMEM((N_CHUNKS, CHUNK), jnp.int32),
            pltpu.VMEM((N_CHUNKS, CHUNK, N_FACTORS), jnp.float32),
            pltpu.VMEM((N_CHUNKS, CHUNK, N_FACTORS), jnp.float32),
            pltpu.VMEM((N_FACTORS,), jnp.float32),
            pltpu.VMEM((LANES,), jnp.float32),
            pltpu.VMEM((B_PER_W,), jnp.float32),
            pltpu.SemaphoreType.DMA,
        ],
        compiler_params=pltpu.CompilerParams(
            needs_layout_passes=False, use_tc_tiling_on_sc=False),
    )(_gmf_body)
    return call(user, item, user_emb, item_emb, h_w, h_b)


def kernel(user, item, user_emb, item_emb, h_w, h_b):
    return _gmf(user, item, user_emb, item_emb, h_w, h_b)
